# baseline (speedup 1.0000x reference)
import jax
import jax.numpy as jnp
from jax.experimental import pallas as pl


def _add_body(x_ref, t_ref, o_ref):
    o_ref[...] = x_ref[...] + t_ref[...]


def kernel(x, table):
    B, T, D = x.shape
    BS = 1024
    xf = x.reshape(B * T, D)
    nseq = T // BS
    out = pl.pallas_call(
        _add_body,
        grid=(B * T // BS,),
        in_specs=[
            pl.BlockSpec((BS, D), lambda s: (s, 0)),
            pl.BlockSpec((BS, D), lambda s, _n=nseq: (s % _n, 0)),
        ],
        out_specs=pl.BlockSpec((BS, D), lambda s: (s, 0)),
        out_shape=jax.ShapeDtypeStruct(xf.shape, xf.dtype),
    )(xf, table)
    return out.reshape(B, T, D)
